# baseline (speedup 1.0000x reference)
import functools

import jax
import jax.numpy as jnp
from jax import lax
from jax.experimental import pallas as pl
from jax.experimental.pallas import tpu as pltpu
from jax.experimental.pallas import tpu_sc as plsc

N = 10000
E = 160000
D = 256
R_MSG = 8

NW = 32
NPT = 313
NPAD = NW * NPT
C = 3200
G = 64
NB = 6
PAD = ((G + 15) // 16) * 16
SENTINEL = -3.0e38


_BN = 1000


def _rne_bf16_bits(v):
    bits = lax.bitcast_convert_type(v, jnp.uint32)
    return (bits + 0x7FFF + ((bits >> 16) & 1)) >> 16


def _msg_table_body(x_ref, w_ref, b_ref, out_ref):
    y = (
        jnp.dot(x_ref[...], w_ref[0], preferred_element_type=jnp.float32)
        + b_ref[0]
    )
    a = _rne_bf16_bits(y[:, : D // 2])
    bb = _rne_bf16_bits(y[:, D // 2 :])
    out_ref[0] = lax.bitcast_convert_type(a | (bb << 16), jnp.int32)


def _msg_table(x, w8, b8):
    nb = N // _BN
    return pl.pallas_call(
        _msg_table_body,
        grid=(nb, R_MSG),
        in_specs=[
            pl.BlockSpec((_BN, D), lambda i, r: (i, 0)),
            pl.BlockSpec((1, D, D), lambda i, r: (r, 0, 0)),
            pl.BlockSpec((1, 1, D), lambda i, r: (r, 0, 0)),
        ],
        out_specs=pl.BlockSpec((1, _BN, D // 2), lambda i, r: (r, i, 0)),
        out_shape=jax.ShapeDtypeStruct((R_MSG, N, D // 2), jnp.int32),
    )(x, w8, b8)


M = 4096
DRAIN_T = M - C - PAD


def _sc_agg_body(
    y_hbm, e_hbm, out_hbm, ebuf, flatm, gidx, acc, rows,
    s0, s1, s2, s3, s4, s5, s6, s7
):
    cid = lax.axis_index("c")
    sid = lax.axis_index("s")
    wid = sid * 2 + cid
    lo = wid * NPT

    neg = jnp.full((16,), -10354847, dtype=jnp.int32)

    def init_body(i, _):
        acc[pl.ds(i * 16, 16)] = neg
        return 0

    lax.fori_loop(0, (NPT * D // 2) // 16, init_body, 0)

    sems = (s0, s1, s2, s3, s6, s7)
    esems = (s4, s5)

    def fire_chunk(ci):
        src = e_hbm.at[:, pl.ds(ci * C, C)]
        for p in range(2):

            @pl.when(ci % 2 == p)
            def _(p=p):
                pltpu.async_copy(src, ebuf.at[p], esems[p])

    def wait_chunk(ci):
        dummy = e_hbm.at[:, pl.ds(0, C)]
        for p in range(2):

            @pl.when(ci % 2 == p)
            def _(p=p):
                pltpu.make_async_copy(dummy, ebuf.at[p], esems[p]).wait()

    def fire(si):
        for p in range(NB):

            @pl.when(si % NB == p)
            def _(p=p):
                for i in range(G // 16):
                    v = flatm[pl.ds(si * G + i * 16, 16)]
                    gidx[p, pl.ds(i * 16, 16)] = v >> 9
                pltpu.async_copy(
                    y_hbm.at[gidx.at[p]], rows.at[p], sems[p]
                )

    def wait(si):
        dummy = y_hbm.at[gidx.at[0]]
        for p in range(NB):

            @pl.when(si % NB == p)
            def _(p=p):
                pltpu.make_async_copy(dummy, rows.at[p], sems[p]).wait()

    def drain(mcnt):
        zero16 = jnp.zeros((16,), dtype=jnp.int32)
        for i in range(PAD // 16):
            flatm[pl.ds(mcnt + i * 16, 16)] = zero16

        nsub = (mcnt + (G - 1)) // G

        @pl.when(nsub > 0)
        def _():
            for k in range(NB - 1):

                @pl.when(k < nsub)
                def _(k=k):
                    fire(k)

            def sub(si, _):
                @pl.when(si + (NB - 1) < nsub)
                def _():
                    fire(si + (NB - 1))

                wait(si)
                p = si % NB
                off = si * G
                m_here = jnp.minimum(mcnt - off, G)

                def edge(j, _):
                    dl = flatm[pl.ds(off + j, 16)][0] & 511
                    rb = dl * (D // 2)
                    for k in range(D // 32):
                        a = plsc.bitcast(
                            acc[pl.ds(rb + k * 16, 16)], jnp.bfloat16
                        )
                        v = plsc.bitcast(
                            rows[p, j, pl.ds(k * 16, 16)], jnp.bfloat16
                        )
                        mx = jnp.maximum(a, v)
                        acc[pl.ds(rb + k * 16, 16)] = plsc.bitcast(
                            mx, jnp.int32
                        )
                    return 0

                lax.fori_loop(0, m_here, edge, 0)
                return 0

            lax.fori_loop(0, nsub, sub, 0)

    def chunk_body(ci, mcnt):
        @pl.when(ci + 1 < E // C)
        def _():
            fire_chunk(ci + 1)

        wait_chunk(ci)
        pe = ci % 2

        def filt(i, cnt):
            s = ebuf[pe, 0, pl.ds(i * 16, 16)]
            d = ebuf[pe, 1, pl.ds(i * 16, 16)]
            r = ebuf[pe, 2, pl.ds(i * 16, 16)]
            dl = d - lo
            m = (dl >= 0) & (dl < NPT)
            packed = (r * N + s) * 512 + dl
            cs = plsc.cumsum(m.astype(jnp.int32))
            pos = (cs - 1) + cnt
            plsc.store_scatter(flatm, [pos], packed, mask=m)
            return cnt + cs[15]

        mcnt = lax.fori_loop(0, C // 16, filt, mcnt)

        @pl.when(mcnt >= DRAIN_T)
        def _():
            drain(mcnt)

        return jnp.where(mcnt >= DRAIN_T, 0, mcnt)

    fire_chunk(0)
    mcnt = lax.fori_loop(0, E // C, chunk_body, 0)
    drain(mcnt)

    pltpu.sync_copy(
        acc, out_hbm.at[pl.ds(lo * (D // 2), NPT * (D // 2))]
    )


def _sc_agg(yf, estack):
    mesh = plsc.VectorSubcoreMesh(core_axis_name="c", subcore_axis_name="s")
    kfn = functools.partial(
        pl.kernel,
        out_type=jax.ShapeDtypeStruct((NPAD * D // 2,), jnp.int32),
        mesh=mesh,
        compiler_params=pltpu.CompilerParams(needs_layout_passes=False),
        scratch_types=[
            pltpu.VMEM((2, 3, C), jnp.int32),
            pltpu.VMEM((M,), jnp.int32),
            pltpu.VMEM((NB, G), jnp.int32),
            pltpu.VMEM((NPT * D // 2,), jnp.int32),
            pltpu.VMEM((NB, G, D // 2), jnp.int32),
            pltpu.SemaphoreType.DMA,
            pltpu.SemaphoreType.DMA,
            pltpu.SemaphoreType.DMA,
            pltpu.SemaphoreType.DMA,
            pltpu.SemaphoreType.DMA,
            pltpu.SemaphoreType.DMA,
            pltpu.SemaphoreType.DMA,
            pltpu.SemaphoreType.DMA,
        ],
    )(_sc_agg_body)
    return kfn(yf, estack)


def _apply_body(x_ref, w_ref, b_ref, agg_ref, out_ref):
    a = agg_ref[...].astype(jnp.float32)
    a = jnp.where(a > -1.0e37, a, 0.0)
    out_ref[...] = (
        jnp.dot(x_ref[...], w_ref[...], preferred_element_type=jnp.float32)
        + b_ref[0][None, :]
        + a
    )


def _apply(x, w_self, b_self, agg):
    nb = N // _BN
    return pl.pallas_call(
        _apply_body,
        grid=(nb,),
        in_specs=[
            pl.BlockSpec((_BN, D), lambda i: (i, 0)),
            pl.BlockSpec((D, D), lambda i: (0, 0)),
            pl.BlockSpec((1, D), lambda i: (0, 0)),
            pl.BlockSpec((_BN, D), lambda i: (i, 0)),
        ],
        out_specs=pl.BlockSpec((_BN, D), lambda i: (i, 0)),
        out_shape=jax.ShapeDtypeStruct((N, D), jnp.float32),
    )(x, w_self, b_self, agg)


def kernel(x, edge_index, rel_type, W, b):
    y = _msg_table(x, W[:R_MSG], b[:R_MSG].reshape(R_MSG, 1, D))
    yf = y.reshape(R_MSG * N, D // 2)
    estack = jnp.concatenate([edge_index, rel_type[None, :]], axis=0)
    agg_flat = _sc_agg(yf, estack)
    ab = lax.bitcast_convert_type(
        agg_flat.reshape(NPAD, D // 2), jnp.bfloat16
    )
    agg = jnp.concatenate([ab[:, :, 0], ab[:, :, 1]], axis=1)
    return _apply(x, W[R_MSG], b[R_MSG].reshape(1, D), agg)
